# Initial kernel scaffold; baseline (speedup 1.0000x reference)
import jax
import jax.numpy as jnp
from jax.experimental import pallas as pl


def kernel(x, edge_index, edge_type, W_rel, W_root, b1, Wg_root, Wg_nbr, b2):
    raise NotImplementedError("write your pallas kernel here")


# speedup vs baseline: 12.7495x; 12.7495x over previous
import functools

import jax
import jax.numpy as jnp
from jax import lax
from jax.experimental import pallas as pl
from jax.experimental.pallas import tpu as pltpu
from jax.experimental.pallas import tpu_sc as plsc

N = 10000
E = 320000
D = 128
H = 64
R = 3

NC = 2
NS = 16
NW = NC * NS
EPW = E // NW
CH = 80
NCHUNK = EPW // CH

_MESH = dict(core_axis_name="c", subcore_axis_name="s", num_cores=NC,
             num_subcores=NS)
_SC_PARAMS = pltpu.CompilerParams(use_tc_tiling_on_sc=False,
                                  needs_layout_passes=False)


def _tc_rel_transform(x, W_rel):
    def body(x_ref, w_ref, o_ref):
        xb = x_ref[...]
        for r in range(R):
            o_ref[r] = jnp.dot(xb, w_ref[r], preferred_element_type=jnp.float32)

    return pl.pallas_call(
        body,
        grid=(10,),
        in_specs=[pl.BlockSpec((N // 10, D), lambda i: (i, 0)),
                  pl.BlockSpec((R, D, H), lambda i: (0, 0, 0))],
        out_specs=pl.BlockSpec((R, N // 10, H), lambda i: (0, i, 0)),
        out_shape=jax.ShapeDtypeStruct((R, N, H), jnp.float32),
    )(x, W_rel)


def _tc_mid(parts, hist, x, W_root, b1, Wg_nbr):
    def body(p_ref, h_ref, x_ref, wr_ref, b1_ref, wn_ref, oh_ref, ohw_ref):
        cnt = jnp.sum(h_ref[...], axis=0)
        norm = 1.0 / jnp.maximum(cnt, 1.0)
        p = p_ref[0] + p_ref[1]
        agg = (p[0] * norm[:, 0:1] + p[1] * norm[:, 1:2] + p[2] * norm[:, 2:3])
        h = agg + jnp.dot(x_ref[...], wr_ref[...],
                          preferred_element_type=jnp.float32) + b1_ref[...]
        h = jnp.maximum(h, 0.0)
        oh_ref[...] = h
        ohw_ref[...] = jnp.dot(h, wn_ref[...],
                               preferred_element_type=jnp.float32)

    B = N // 10
    return pl.pallas_call(
        body,
        grid=(10,),
        in_specs=[pl.BlockSpec((NC, R, B, H), lambda i: (0, 0, i, 0)),
                  pl.BlockSpec((NW, B, R), lambda i: (0, i, 0)),
                  pl.BlockSpec((B, D), lambda i: (i, 0)),
                  pl.BlockSpec((D, H), lambda i: (0, 0)),
                  pl.BlockSpec((1, H), lambda i: (0, 0)),
                  pl.BlockSpec((H, H), lambda i: (0, 0))],
        out_specs=[pl.BlockSpec((B, H), lambda i: (i, 0)),
                   pl.BlockSpec((B, H), lambda i: (i, 0))],
        out_shape=[jax.ShapeDtypeStruct((N, H), jnp.float32),
                   jax.ShapeDtypeStruct((N, H), jnp.float32)],
    )(parts, hist, x, W_root, b1, Wg_nbr)


def _tc_post(x, h, parts2, Wg_root, b2):
    def body(x_ref, h_ref, q_ref, wg_ref, b2_ref, o_ref):
        nbrw = q_ref[0] + q_ref[1]
        h2 = jnp.dot(h_ref[...], wg_ref[...],
                     preferred_element_type=jnp.float32) + nbrw + b2_ref[...]
        h2 = jnp.maximum(h2, 0.0)
        o_ref[...] = jnp.concatenate([x_ref[...], h2], axis=1)

    B = N // 10
    return pl.pallas_call(
        body,
        grid=(10,),
        in_specs=[pl.BlockSpec((B, D), lambda i: (i, 0)),
                  pl.BlockSpec((B, H), lambda i: (i, 0)),
                  pl.BlockSpec((NC, B, H), lambda i: (0, i, 0)),
                  pl.BlockSpec((H, H), lambda i: (0, 0)),
                  pl.BlockSpec((1, H), lambda i: (0, 0))],
        out_specs=pl.BlockSpec((B, D + H), lambda i: (i, 0)),
        out_shape=jax.ShapeDtypeStruct((N, D + H), jnp.float32),
    )(x, h, parts2, Wg_root, b2)


def _sc_counts(hidx, zhist):
    @functools.partial(
        pl.kernel,
        out_type=jax.ShapeDtypeStruct((NW * R * N,), jnp.float32),
        mesh=plsc.VectorSubcoreMesh(**_MESH),
        compiler_params=_SC_PARAMS,
        scratch_types=[
            pltpu.VMEM((R * N,), jnp.float32),
            pltpu.VMEM((CH,), jnp.int32),
        ],
    )
    def k(hidx_hbm, zh_hbm, hist_hbm, hist_v, idx_v):
        c = lax.axis_index("c")
        s = lax.axis_index("s")
        wid = c * NS + s
        pltpu.sync_copy(zh_hbm, hist_v)
        ones = jnp.ones((16,), jnp.float32)

        def chunk(j, carry):
            pltpu.sync_copy(hidx_hbm.at[pl.ds(wid * EPW + j * CH, CH)], idx_v)
            for g in range(CH // 16):
                plsc.addupdate_scatter(hist_v, [idx_v[pl.ds(g * 16, 16)]],
                                       ones)
            return carry

        lax.fori_loop(0, NCHUNK, chunk, 0)
        pltpu.sync_copy(hist_v, hist_hbm.at[pl.ds(wid * (R * N), R * N)])

    return k(hidx, zhist)


def _sc_edge_pass(table, gsrc, gdst, zrows, n_rows):
    rpt = n_rows // NS

    @functools.partial(
        pl.kernel,
        out_type=jax.ShapeDtypeStruct((NC, NS, rpt, H), jnp.float32),
        mesh=plsc.VectorSubcoreMesh(**_MESH),
        compiler_params=_SC_PARAMS,
        scratch_types=[
            pltpu.VMEM_SHARED((n_rows, H), jnp.float32),
            pltpu.VMEM((CH,), jnp.int32),
            pltpu.VMEM((CH,), jnp.int32),
            pltpu.VMEM((CH, H), jnp.float32),
        ],
    )
    def k(tab_hbm, gsrc_hbm, gdst_hbm, z_hbm, parts_hbm,
          acc, gsrc_v, gdst_v, rows_v):
        c = lax.axis_index("c")
        s = lax.axis_index("s")
        wid = c * NS + s
        pltpu.sync_copy(z_hbm, acc.at[pl.ds(s * rpt, rpt)])
        plsc.subcore_barrier()

        def chunk(j, carry):
            base = wid * EPW + j * CH
            pltpu.sync_copy(gsrc_hbm.at[pl.ds(base, CH)], gsrc_v)
            pltpu.sync_copy(gdst_hbm.at[pl.ds(base, CH)], gdst_v)
            pltpu.sync_copy(tab_hbm.at[gsrc_v], rows_v)
            pltpu.sync_copy(rows_v, acc.at[gdst_v], add=True)
            return carry

        lax.fori_loop(0, NCHUNK, chunk, 0)
        plsc.subcore_barrier()
        pltpu.sync_copy(acc.at[pl.ds(s * rpt, rpt)], parts_hbm.at[c, s])

    return k(table, gsrc, gdst, zrows)


def kernel(x, edge_index, edge_type, W_rel, W_root, b1, Wg_root, Wg_nbr, b2):
    src = edge_index[0]
    dst = edge_index[1]
    gsrc = edge_type * N + src
    gdst = edge_type * N + dst
    hidx = dst * R + edge_type

    zrows1 = jnp.zeros((R * N // NS, H), jnp.float32)
    zrows2 = jnp.zeros((N // NS, H), jnp.float32)
    zhist = jnp.zeros((R * N,), jnp.float32)

    hist = _sc_counts(hidx, zhist)
    xr = _tc_rel_transform(x, W_rel)
    parts = _sc_edge_pass(xr.reshape(R * N, H), gsrc, gdst, zrows1, R * N)
    h, hw = _tc_mid(parts.reshape(NC, R, N, H), hist.reshape(NW, N, R),
                    x, W_root, b1.reshape(1, H), Wg_nbr)
    parts2 = _sc_edge_pass(hw, src, dst, zrows2, N)
    out = _tc_post(x, h, parts2.reshape(NC, N, H), Wg_root, b2.reshape(1, H))
    return out
